# baseline (speedup 1.0000x reference)
import functools

import jax
import jax.numpy as jnp
from jax import lax
from jax.experimental import pallas as pl
from jax.experimental.pallas import tpu as pltpu
from jax.experimental.pallas import tpu_sc as plsc

N = 1000000
H = 64
FD = 32
B = 16384

NC = 2
NS = 16
NW = NC * NS
TOT = 2 * B
ROWS_PER_W = TOT // NW
GRP = 16
NGRP = ROWS_PER_W // GRP


def _sc_gather_kernel(mem1, idx_hbm, out1, idx_v, rows_v, sem):
    wid = lax.axis_index("s") * NC + lax.axis_index("c")
    base = wid * ROWS_PER_W
    pltpu.sync_copy(idx_hbm.at[pl.ds(base, ROWS_PER_W)], idx_v)

    def chunk(c, carry):
        rvec = idx_v[pl.ds(c * GRP, GRP)]
        for j in range(GRP):
            r = rvec[j]
            pltpu.async_copy(
                mem1.at[pl.ds(r * H, H)],
                rows_v.at[pl.ds((c * GRP + j) * H, H)], sem)
        return carry

    lax.fori_loop(0, NGRP, chunk, 0)
    pltpu.make_async_copy(
        mem1.at[pl.ds(0, ROWS_PER_W * H)], rows_v, sem).wait()
    pltpu.sync_copy(rows_v, out1.at[pl.ds(base * H, ROWS_PER_W * H)])


def _sc_gather(mem1, idx):
    mesh = plsc.VectorSubcoreMesh(core_axis_name="c", subcore_axis_name="s")
    return pl.kernel(
        _sc_gather_kernel,
        mesh=mesh,
        compiler_params=pltpu.CompilerParams(use_tc_tiling_on_sc=False),
        out_type=jax.ShapeDtypeStruct((TOT * H,), jnp.float32),
        scratch_types=[
            pltpu.VMEM((ROWS_PER_W,), jnp.int32),
            pltpu.VMEM((ROWS_PER_W * H,), jnp.float32),
            pltpu.SemaphoreType.DMA,
        ],
    )(mem1, idx)


BLK = 2048
NBLK = B // BLK


def _tc_mlp_kernel(ms_ref, md_ref, f_ref, w1s_ref, w1d_ref, w1f_ref,
                   b1_ref, w2_ref, b2_ref, out_ref):
    acc = jnp.dot(ms_ref[:], w1s_ref[:], preferred_element_type=jnp.float32)
    acc += jnp.dot(md_ref[:], w1d_ref[:], preferred_element_type=jnp.float32)
    acc += jnp.dot(f_ref[:], w1f_ref[:], preferred_element_type=jnp.float32)
    h = jnp.maximum(acc + b1_ref[:], 0.0)
    out_ref[:] = jnp.sum(h * w2_ref[:], axis=1) + b2_ref[0, 0]


def _tc_mlp(gathered, feats, w1sT, w1dT, w1fT, b1, w2, b2):
    return pl.pallas_call(
        _tc_mlp_kernel,
        grid=(NBLK,),
        in_specs=[
            pl.BlockSpec((BLK, H), lambda i: (i, 0)),
            pl.BlockSpec((BLK, H), lambda i: (i + NBLK, 0)),
            pl.BlockSpec((BLK, FD), lambda i: (i, 0)),
            pl.BlockSpec((H, H), lambda i: (0, 0)),
            pl.BlockSpec((H, H), lambda i: (0, 0)),
            pl.BlockSpec((FD, H), lambda i: (0, 0)),
            pl.BlockSpec((1, H), lambda i: (0, 0)),
            pl.BlockSpec((1, H), lambda i: (0, 0)),
            pl.BlockSpec((1, 1), lambda i: (0, 0)),
        ],
        out_specs=pl.BlockSpec((BLK,), lambda i: (i,)),
        out_shape=jax.ShapeDtypeStruct((B,), jnp.float32),
    )(gathered, gathered, feats, w1sT, w1dT, w1fT, b1, w2, b2)


def kernel(s, d, feats, mem, w1, b1, w2, b2):
    idx = jnp.concatenate([s.astype(jnp.int32), d.astype(jnp.int32)])
    mem1 = mem.reshape(N * H)
    gathered = _sc_gather(mem1, idx).reshape(TOT, H)
    w1sT = w1[:, :H].T
    w1dT = w1[:, H:2 * H].T
    w1fT = w1[:, 2 * H:].T
    return _tc_mlp(gathered, feats, w1sT, w1dT, w1fT,
                   b1.reshape(1, H), w2, b2.reshape(1, 1))
